# baseline (speedup 1.0000x reference)
import functools

import jax
import jax.numpy as jnp
from jax import lax
from jax.experimental import pallas as pl
from jax.experimental.pallas import tpu as pltpu


def _fused_mlp_kernel(x_ref, w1_ref, g_ref, beta_ref, w2_ref, b2_ref,
                      o_ref, hn_ref, *, eps, inv_b, n_h, t_h):
    j = pl.program_id(1)

    @pl.when(j < n_h)
    def _hidden_tile():
        h = jnp.dot(x_ref[...], w1_ref[...],
                    preferred_element_type=jnp.float32)
        mean = jnp.sum(h, axis=0, keepdims=True) * inv_b
        var = jnp.sum(h * h, axis=0, keepdims=True) * inv_b - mean * mean
        a = g_ref[...] * lax.rsqrt(jnp.maximum(var, 0.0) + eps)
        c = beta_ref[...] - mean * a
        col = pl.multiple_of(j * t_h, t_h)
        hn_ref[:, pl.ds(col, t_h)] = jnp.maximum(h * a + c, 0.0)

    @pl.when(j >= n_h)
    def _out_tile():
        o_ref[...] = (jnp.dot(hn_ref[...], w2_ref[...],
                              preferred_element_type=jnp.float32)
                      + b2_ref[...])


def kernel(w1, b1, gamma, beta, w2, b2, x):
    del b1
    B, in_dim = x.shape
    hidden = w1.shape[1]
    out_dim = w2.shape[1]
    eps = 1e-5

    g2 = gamma.reshape(1, hidden)
    beta2 = beta.reshape(1, hidden)
    b2_2 = b2.reshape(1, out_dim)

    t_h = 256 if hidden % 256 == 0 else hidden
    n_h = hidden // t_h
    t_n = 256 if out_dim % 512 == 0 else out_dim
    ncore = 2 if out_dim % (2 * t_n) == 0 else 1
    n_n = out_dim // (ncore * t_n)
    steps = n_h + n_n

    def w1_idx(i, j):
        return (0, jnp.minimum(j, n_h - 1))

    def bn_idx(i, j):
        return (0, jnp.minimum(j, n_h - 1))

    def w2_idx(i, j):
        return (0, i * n_n + jnp.clip(j - n_h, 0, n_n - 1))

    def out_idx(i, j):
        return (0, i * n_n + jnp.clip(j - n_h, 0, n_n - 1))

    body = functools.partial(_fused_mlp_kernel, eps=eps, inv_b=1.0 / B,
                             n_h=n_h, t_h=t_h)
    return pl.pallas_call(
        body,
        grid=(ncore, steps),
        in_specs=[
            pl.BlockSpec((B, in_dim), lambda i, j: (0, 0)),
            pl.BlockSpec((in_dim, t_h), w1_idx),
            pl.BlockSpec((1, t_h), bn_idx),
            pl.BlockSpec((1, t_h), bn_idx),
            pl.BlockSpec((hidden, t_n), w2_idx),
            pl.BlockSpec((1, t_n), w2_idx),
        ],
        out_specs=pl.BlockSpec((B, t_n), out_idx),
        out_shape=jax.ShapeDtypeStruct((B, out_dim), jnp.float32),
        scratch_shapes=[pltpu.VMEM((B, hidden), jnp.float32)],
        compiler_params=pltpu.CompilerParams(
            dimension_semantics=("parallel", "arbitrary")),
        cost_estimate=pl.CostEstimate(
            flops=2 * B * in_dim * hidden * ncore
            + 2 * B * hidden * out_dim,
            transcendentals=hidden * ncore,
            bytes_accessed=(2 * B * in_dim + 2 * in_dim * hidden
                            + hidden * out_dim + B * out_dim) * 4,
        ),
    )(x, w1, g2, beta2, w2, b2_2)


# speedup vs baseline: 1.4820x; 1.4820x over previous
import functools

import jax
import jax.numpy as jnp
from jax import lax
from jax.experimental import pallas as pl
from jax.experimental.pallas import tpu as pltpu


def _fused_mlp_kernel(x_ref, w1_ref, g_ref, beta_ref, w2_ref, b2_ref,
                      o_ref, hn_ref, *, eps, inv_b, n_h, t_h):
    j = pl.program_id(0)

    @pl.when(j < n_h)
    def _hidden_tile():
        h = jnp.dot(x_ref[...], w1_ref[...],
                    preferred_element_type=jnp.float32)
        mean = jnp.sum(h, axis=0, keepdims=True) * inv_b
        var = jnp.sum(h * h, axis=0, keepdims=True) * inv_b - mean * mean
        a = g_ref[...] * lax.rsqrt(jnp.maximum(var, 0.0) + eps)
        c = beta_ref[...] - mean * a
        col = pl.multiple_of(j * t_h, t_h)
        hn_ref[:, pl.ds(col, t_h)] = jnp.maximum(h * a + c, 0.0)

    @pl.when(j >= n_h)
    def _out_tile():
        o_ref[...] = (jnp.dot(hn_ref[...], w2_ref[...],
                              preferred_element_type=jnp.float32)
                      + b2_ref[...])


def kernel(w1, b1, gamma, beta, w2, b2, x):
    del b1
    B, in_dim = x.shape
    hidden = w1.shape[1]
    out_dim = w2.shape[1]
    eps = 1e-5

    g2 = gamma.reshape(1, hidden)
    beta2 = beta.reshape(1, hidden)
    b2_2 = b2.reshape(1, out_dim)

    t_h = 256 if hidden % 256 == 0 else hidden
    n_h = hidden // t_h
    t_n = 256 if out_dim % 256 == 0 else out_dim
    n_n = out_dim // t_n
    steps = n_h + n_n

    def w1_idx(j):
        return (0, jnp.minimum(j, n_h - 1))

    def w2_idx(j):
        return (0, jnp.clip(j - n_h, 0, n_n - 1))

    body = functools.partial(_fused_mlp_kernel, eps=eps, inv_b=1.0 / B,
                             n_h=n_h, t_h=t_h)
    return pl.pallas_call(
        body,
        grid=(steps,),
        in_specs=[
            pl.BlockSpec((B, in_dim), lambda j: (0, 0)),
            pl.BlockSpec((in_dim, t_h), w1_idx),
            pl.BlockSpec((1, t_h), w1_idx),
            pl.BlockSpec((1, t_h), w1_idx),
            pl.BlockSpec((hidden, t_n), w2_idx),
            pl.BlockSpec((1, t_n), w2_idx),
        ],
        out_specs=pl.BlockSpec((B, t_n), w2_idx),
        out_shape=jax.ShapeDtypeStruct((B, out_dim), jnp.float32),
        scratch_shapes=[pltpu.VMEM((B, hidden), jnp.float32)],
        compiler_params=pltpu.CompilerParams(
            dimension_semantics=("arbitrary",)),
        cost_estimate=pl.CostEstimate(
            flops=2 * B * in_dim * hidden + 2 * B * hidden * out_dim,
            transcendentals=hidden,
            bytes_accessed=(B * in_dim + in_dim * hidden
                            + hidden * out_dim + B * out_dim) * 4,
        ),
    )(x, w1, g2, beta2, w2, b2_2)
